# baseline (speedup 1.0000x reference)
import functools

import jax
import jax.numpy as jnp
from jax import lax
from jax.experimental import pallas as pl
from jax.experimental.pallas import tpu as pltpu
from jax.experimental.pallas import tpu_sc as plsc

_B, _BAG, _L = 1024, 4, 50
_V, _D = 1000000, 64
_MD, _H = 128, 128
_S = _B * _BAG
_NC, _NS = 2, 16
_NW = _NC * _NS
_SW = _S // _NW
_Z = _V // 2


def _embed_sum(pairs, idx_l, idx_r):
    mesh = plsc.VectorSubcoreMesh(core_axis_name="c", subcore_axis_name="s")

    @functools.partial(
        pl.kernel, mesh=mesh,
        out_type=jax.ShapeDtypeStruct((_S, 2 * _D), jnp.float32),
        scratch_types=[
            pltpu.VMEM((_L, _SW), jnp.int32),
            pltpu.VMEM((_L, _SW), jnp.int32),
            pltpu.VMEM((_SW, 2 * _D), jnp.float32),
            pltpu.VMEM((_SW, 2 * _D), jnp.float32),
            pltpu.SemaphoreType.DMA,
        ],
    )
    def k(pairs_hbm, idxl_hbm, idxr_hbm, out_hbm, il_v, ir_v, al_v, ar_v,
          sem):
        cid = lax.axis_index("c")
        sid = lax.axis_index("s")
        wid = sid * _NC + cid

        pltpu.sync_copy(idxl_hbm.at[wid], il_v)
        pltpu.sync_copy(idxr_hbm.at[wid], ir_v)

        zeros = jnp.zeros((16,), jnp.float32)

        def zbody(i, carry):
            for j in range(2 * _D // 16):
                al_v[i, pl.ds(j * 16, 16)] = zeros
                ar_v[i, pl.ds(j * 16, 16)] = zeros
            return carry
        lax.fori_loop(0, _SW, zbody, 0)

        copies = [
            pltpu.async_copy(pairs_hbm.at[il_v.at[j]], al_v, sem, add=True)
            for j in range(_L)
        ] + [
            pltpu.async_copy(pairs_hbm.at[ir_v.at[j]], ar_v, sem, add=True)
            for j in range(_L)
        ]
        for c in copies:
            c.wait()

        def cbody(i, carry):
            for j in range(_D // 16):
                al_v[i, pl.ds(j * 16, 16)] += ar_v[i, pl.ds(_D + j * 16, 16)]
            return carry
        lax.fori_loop(0, _SW, cbody, 0)

        pltpu.sync_copy(al_v, out_hbm.at[pl.ds(wid * _SW, _SW)])

    return k(pairs, idx_l, idx_r)


def _mlp_body(bag_ref, mr_ref, w1a_ref, w1b_ref, b1_ref, w2_ref, b2_ref,
              o_ref):
    h = jnp.dot(bag_ref[:, :_D], w1a_ref[...],
                preferred_element_type=jnp.float32)
    h = h + jnp.dot(mr_ref[...], w1b_ref[...],
                    preferred_element_type=jnp.float32)
    h = jnp.maximum(h + b1_ref[...], 0.0)
    o = jnp.dot(h, w2_ref[...], preferred_element_type=jnp.float32)
    o_ref[...] = jnp.maximum(o + b2_ref[...], 0.0)


def _mlp(bag, mr, w1a, w1b, b1, w2, b2):
    R = 512
    return pl.pallas_call(
        _mlp_body,
        grid=(_S // R,),
        in_specs=[
            pl.BlockSpec((R, 2 * _D), lambda i: (i, 0)),
            pl.BlockSpec((R, _MD), lambda i: (i, 0)),
            pl.BlockSpec((_D, 2 * _H), lambda i: (0, 0)),
            pl.BlockSpec((_MD, 2 * _H), lambda i: (0, 0)),
            pl.BlockSpec((1, 2 * _H), lambda i: (0, 0)),
            pl.BlockSpec((2 * _H, _H), lambda i: (0, 0)),
            pl.BlockSpec((1, _H), lambda i: (0, 0)),
        ],
        out_specs=pl.BlockSpec((R, _H), lambda i: (i, 0)),
        out_shape=jax.ShapeDtypeStruct((_S, _H), jnp.float32),
    )(bag, mr, w1a, w1b, b1, w2, b2)


def kernel(sentences, mention_rep, table, W1, b1, W2, b2):
    pairs = jnp.concatenate([table[0::2], table[1::2]], axis=1)
    pairs = jnp.pad(pairs, ((0, 8), (0, 0)))
    sent_t = sentences.reshape(_NW, _SW, _L).transpose(0, 2, 1)
    u = sent_t // 2
    even = (sent_t % 2) == 0
    idx_l = jnp.where(even, u, _Z)
    idx_r = jnp.where(even, _Z, u)
    bag_sum = _embed_sum(pairs, idx_l, idx_r)
    w1a = W1[:_D] * jnp.float32(1.0 / _L)
    w1b = W1[_D:]
    mr = mention_rep.reshape(_S, _MD)
    return _mlp(bag_sum, mr, w1a, w1b, b1.reshape(1, 2 * _H), W2,
                b2.reshape(1, _H))


# speedup vs baseline: 1.9315x; 1.9315x over previous
import functools

import jax
import jax.numpy as jnp
from jax import lax
from jax.experimental import pallas as pl
from jax.experimental.pallas import tpu as pltpu
from jax.experimental.pallas import tpu_sc as plsc

_B, _BAG, _L = 1024, 4, 50
_V, _D = 1000000, 64
_MD, _H = 128, 128
_S = _B * _BAG
_NC, _NS = 2, 16
_NW = _NC * _NS
_SW = _S // _NW
_Z = _V // 2


def _embed_sum(pairs, idx_l, idx_r):
    mesh = plsc.VectorSubcoreMesh(core_axis_name="c", subcore_axis_name="s")

    @functools.partial(
        pl.kernel, mesh=mesh,
        out_type=jax.ShapeDtypeStruct((_S, 2 * _D), jnp.float32),
        scratch_types=[
            pltpu.VMEM((_L, _SW), jnp.int32),
            pltpu.VMEM((_L, _SW), jnp.int32),
            pltpu.VMEM((_SW, 2 * _D), jnp.float32),
            pltpu.VMEM((_SW, 2 * _D), jnp.float32),
            pltpu.SemaphoreType.DMA,
        ],
    )
    def k(pairs_hbm, idxl_hbm, idxr_hbm, out_hbm, il_v, ir_v, al_v, ar_v,
          sem):
        cid = lax.axis_index("c")
        sid = lax.axis_index("s")
        wid = sid * _NC + cid

        pltpu.sync_copy(idxl_hbm.at[wid], il_v)
        pltpu.sync_copy(idxr_hbm.at[wid], ir_v)

        zeros = jnp.zeros((16,), jnp.float32)

        def zbody(i, carry):
            for j in range(2 * _D // 16):
                al_v[i, pl.ds(j * 16, 16)] = zeros
                ar_v[i, pl.ds(j * 16, 16)] = zeros
            return carry
        lax.fori_loop(0, _SW, zbody, 0)

        copies = [
            pltpu.async_copy(pairs_hbm.at[il_v.at[j]], al_v, sem, add=True)
            for j in range(_L)
        ] + [
            pltpu.async_copy(pairs_hbm.at[ir_v.at[j]], ar_v, sem, add=True)
            for j in range(_L)
        ]
        for c in copies:
            c.wait()

        def cbody(i, carry):
            for j in range(_D // 16):
                al_v[i, pl.ds(j * 16, 16)] += ar_v[i, pl.ds(_D + j * 16, 16)]
            return carry
        lax.fori_loop(0, _SW, cbody, 0)

        pltpu.sync_copy(al_v, out_hbm.at[pl.ds(wid * _SW, _SW)])

    return k(pairs, idx_l, idx_r)


def _mlp_body(bag_ref, mr_ref, w1a_ref, w1b_ref, b1_ref, w2_ref, b2_ref,
              o_ref):
    h = jnp.dot(bag_ref[:, :_D], w1a_ref[...],
                preferred_element_type=jnp.float32)
    h = h + jnp.dot(mr_ref[...], w1b_ref[...],
                    preferred_element_type=jnp.float32)
    h = jnp.maximum(h + b1_ref[...], 0.0)
    o = jnp.dot(h, w2_ref[...], preferred_element_type=jnp.float32)
    o_ref[...] = jnp.maximum(o + b2_ref[...], 0.0)


def _mlp(bag, mr, w1a, w1b, b1, w2, b2):
    R = 512
    return pl.pallas_call(
        _mlp_body,
        grid=(_S // R,),
        in_specs=[
            pl.BlockSpec((R, 2 * _D), lambda i: (i, 0)),
            pl.BlockSpec((R, _MD), lambda i: (i, 0)),
            pl.BlockSpec((_D, 2 * _H), lambda i: (0, 0)),
            pl.BlockSpec((_MD, 2 * _H), lambda i: (0, 0)),
            pl.BlockSpec((1, 2 * _H), lambda i: (0, 0)),
            pl.BlockSpec((2 * _H, _H), lambda i: (0, 0)),
            pl.BlockSpec((1, _H), lambda i: (0, 0)),
        ],
        out_specs=pl.BlockSpec((R, _H), lambda i: (i, 0)),
        out_shape=jax.ShapeDtypeStruct((_S, _H), jnp.float32),
    )(bag, mr, w1a, w1b, b1, w2, b2)


def kernel(sentences, mention_rep, table, W1, b1, W2, b2):
    pairs = jnp.pad(table.reshape(_Z, 2 * _D), ((0, 8), (0, 0)))
    sent_t = sentences.reshape(_NW, _SW, _L).transpose(0, 2, 1)
    u = sent_t // 2
    even = (sent_t % 2) == 0
    idx_l = jnp.where(even, u, _Z)
    idx_r = jnp.where(even, _Z, u)
    bag_sum = _embed_sum(pairs, idx_l, idx_r)
    w1a = W1[:_D] * jnp.float32(1.0 / _L)
    w1b = W1[_D:]
    mr = mention_rep.reshape(_S, _MD)
    return _mlp(bag_sum, mr, w1a, w1b, b1.reshape(1, 2 * _H), W2,
                b2.reshape(1, _H))


# speedup vs baseline: 23.0011x; 11.9082x over previous
import functools

import jax
import jax.numpy as jnp
from jax import lax
from jax.experimental import pallas as pl
from jax.experimental.pallas import tpu as pltpu
from jax.experimental.pallas import tpu_sc as plsc

_B, _BAG, _L = 1024, 4, 50
_V, _D = 1000000, 64
_MD, _H = 128, 128
_S = _B * _BAG
_NC, _NS = 2, 16
_NW = _NC * _NS
_SW = _S // _NW
_RB = 25000


def _retype_body(x_ref, o_ref):
    o_ref[...] = jnp.concatenate(
        [x_ref[...], jnp.zeros((_RB, _D), jnp.float32)], axis=1)


def _retype(table):
    return pl.pallas_call(
        _retype_body,
        grid=(_V // _RB,),
        in_specs=[pl.BlockSpec((_RB, _D), lambda i: (i, 0))],
        out_specs=pl.BlockSpec((_RB, 2 * _D), lambda i: (i, 0)),
        out_shape=jax.ShapeDtypeStruct((_V, 2 * _D), jnp.float32),
    )(table)


def _embed_sum(table_w, sent_t):
    mesh = plsc.VectorSubcoreMesh(core_axis_name="c", subcore_axis_name="s")

    @functools.partial(
        pl.kernel, mesh=mesh,
        out_type=jax.ShapeDtypeStruct((_S, 2 * _D), jnp.float32),
        scratch_types=[
            pltpu.VMEM((_L, _SW), jnp.int32),
            pltpu.VMEM((_SW, 2 * _D), jnp.float32),
            pltpu.SemaphoreType.DMA,
        ],
    )
    def k(table_hbm, sent_hbm, out_hbm, idx_v, acc_v, sem):
        cid = lax.axis_index("c")
        sid = lax.axis_index("s")
        wid = sid * _NC + cid

        pltpu.sync_copy(sent_hbm.at[wid], idx_v)

        zeros = jnp.zeros((16,), jnp.float32)

        def zbody(i, carry):
            for j in range(2 * _D // 16):
                acc_v[i, pl.ds(j * 16, 16)] = zeros
            return carry
        lax.fori_loop(0, _SW, zbody, 0)

        copies = [
            pltpu.async_copy(table_hbm.at[idx_v.at[j]], acc_v, sem, add=True)
            for j in range(_L)
        ]
        for c in copies:
            c.wait()

        pltpu.sync_copy(acc_v, out_hbm.at[pl.ds(wid * _SW, _SW)])

    return k(table_w, sent_t)


def _mlp_body(bag_ref, mr_ref, w1a_ref, w1b_ref, b1_ref, w2_ref, b2_ref,
              o_ref):
    h = jnp.dot(bag_ref[:, :_D], w1a_ref[...],
                preferred_element_type=jnp.float32)
    h = h + jnp.dot(mr_ref[...], w1b_ref[...],
                    preferred_element_type=jnp.float32)
    h = jnp.maximum(h + b1_ref[...], 0.0)
    o = jnp.dot(h, w2_ref[...], preferred_element_type=jnp.float32)
    o_ref[...] = jnp.maximum(o + b2_ref[...], 0.0)


def _mlp(bag, mr, w1a, w1b, b1, w2, b2):
    R = 512
    return pl.pallas_call(
        _mlp_body,
        grid=(_S // R,),
        in_specs=[
            pl.BlockSpec((R, 2 * _D), lambda i: (i, 0)),
            pl.BlockSpec((R, _MD), lambda i: (i, 0)),
            pl.BlockSpec((_D, 2 * _H), lambda i: (0, 0)),
            pl.BlockSpec((_MD, 2 * _H), lambda i: (0, 0)),
            pl.BlockSpec((1, 2 * _H), lambda i: (0, 0)),
            pl.BlockSpec((2 * _H, _H), lambda i: (0, 0)),
            pl.BlockSpec((1, _H), lambda i: (0, 0)),
        ],
        out_specs=pl.BlockSpec((R, _H), lambda i: (i, 0)),
        out_shape=jax.ShapeDtypeStruct((_S, _H), jnp.float32),
    )(bag, mr, w1a, w1b, b1, w2, b2)


def kernel(sentences, mention_rep, table, W1, b1, W2, b2):
    table_w = _retype(table)
    sent_t = sentences.reshape(_NW, _SW, _L).transpose(0, 2, 1)
    bag_sum = _embed_sum(table_w, sent_t)
    w1a = W1[:_D] * jnp.float32(1.0 / _L)
    w1b = W1[_D:]
    mr = mention_rep.reshape(_S, _MD)
    return _mlp(bag_sum, mr, w1a, w1b, b1.reshape(1, 2 * _H), W2,
                b2.reshape(1, _H))


# speedup vs baseline: 27.8477x; 1.2107x over previous
import functools

import jax
import jax.numpy as jnp
from jax import lax
from jax.experimental import pallas as pl
from jax.experimental.pallas import tpu as pltpu
from jax.experimental.pallas import tpu_sc as plsc

_B, _BAG, _L = 1024, 4, 50
_V, _D = 1000000, 64
_MD, _H = 128, 128
_S = _B * _BAG
_NC, _NS = 2, 16
_NW = _NC * _NS
_SW = _S // _NW
_RB = 8000


def _retype_body(x_ref, o_ref):
    o_ref[...] = jnp.concatenate(
        [x_ref[...], jnp.zeros((_RB, _D), jnp.float32)], axis=1)


def _retype(table):
    return pl.pallas_call(
        _retype_body,
        grid=(_V // _RB,),
        in_specs=[pl.BlockSpec((_RB, _D), lambda i: (i, 0))],
        out_specs=pl.BlockSpec((_RB, 2 * _D), lambda i: (i, 0)),
        out_shape=jax.ShapeDtypeStruct((_V, 2 * _D), jnp.float32),
    )(table)


def _embed_sum(table_w, sent_t):
    mesh = plsc.VectorSubcoreMesh(core_axis_name="c", subcore_axis_name="s")

    @functools.partial(
        pl.kernel, mesh=mesh,
        out_type=jax.ShapeDtypeStruct((_S, 2 * _D), jnp.float32),
        scratch_types=[
            pltpu.VMEM((_L, _SW), jnp.int32),
            pltpu.VMEM((_SW, 2 * _D), jnp.float32),
            pltpu.SemaphoreType.DMA,
        ],
    )
    def k(table_hbm, sent_hbm, out_hbm, idx_v, acc_v, sem):
        cid = lax.axis_index("c")
        sid = lax.axis_index("s")
        wid = sid * _NC + cid

        pltpu.sync_copy(sent_hbm.at[wid], idx_v)

        zeros = jnp.zeros((16,), jnp.float32)

        def zbody(i, carry):
            for j in range(2 * _D // 16):
                acc_v[i, pl.ds(j * 16, 16)] = zeros
            return carry
        lax.fori_loop(0, _SW, zbody, 0)

        copies = [
            pltpu.async_copy(table_hbm.at[idx_v.at[j]], acc_v, sem, add=True)
            for j in range(_L)
        ]
        for c in copies:
            c.wait()

        pltpu.sync_copy(acc_v, out_hbm.at[pl.ds(wid * _SW, _SW)])

    return k(table_w, sent_t)


def _mlp_body(bag_ref, mr_ref, w1a_ref, w1b_ref, b1_ref, w2_ref, b2_ref,
              o_ref):
    h = jnp.dot(bag_ref[:, :_D], w1a_ref[...],
                preferred_element_type=jnp.float32)
    h = h + jnp.dot(mr_ref[...], w1b_ref[...],
                    preferred_element_type=jnp.float32)
    h = jnp.maximum(h + b1_ref[...], 0.0)
    o = jnp.dot(h, w2_ref[...], preferred_element_type=jnp.float32)
    o_ref[...] = jnp.maximum(o + b2_ref[...], 0.0)


def _mlp(bag, mr, w1a, w1b, b1, w2, b2):
    R = 512
    return pl.pallas_call(
        _mlp_body,
        grid=(_S // R,),
        in_specs=[
            pl.BlockSpec((R, 2 * _D), lambda i: (i, 0)),
            pl.BlockSpec((R, _MD), lambda i: (i, 0)),
            pl.BlockSpec((_D, 2 * _H), lambda i: (0, 0)),
            pl.BlockSpec((_MD, 2 * _H), lambda i: (0, 0)),
            pl.BlockSpec((1, 2 * _H), lambda i: (0, 0)),
            pl.BlockSpec((2 * _H, _H), lambda i: (0, 0)),
            pl.BlockSpec((1, _H), lambda i: (0, 0)),
        ],
        out_specs=pl.BlockSpec((R, _H), lambda i: (i, 0)),
        out_shape=jax.ShapeDtypeStruct((_S, _H), jnp.float32),
    )(bag, mr, w1a, w1b, b1, w2, b2)


def kernel(sentences, mention_rep, table, W1, b1, W2, b2):
    table_w = jnp.pad(table, ((0, 0), (0, _D)))
    sent_t = sentences.reshape(_NW, _SW, _L).transpose(0, 2, 1)
    bag_sum = _embed_sum(table_w, sent_t)
    w1a = W1[:_D] * jnp.float32(1.0 / _L)
    w1b = W1[_D:]
    mr = mention_rep.reshape(_S, _MD)
    return _mlp(bag_sum, mr, w1a, w1b, b1.reshape(1, 2 * _H), W2,
                b2.reshape(1, _H))


# speedup vs baseline: 27.9769x; 1.0046x over previous
import functools

import jax
import jax.numpy as jnp
from jax import lax
from jax.experimental import pallas as pl
from jax.experimental.pallas import tpu as pltpu
from jax.experimental.pallas import tpu_sc as plsc

_B, _BAG, _L = 1024, 4, 50
_V, _D = 1000000, 64
_MD, _H = 128, 128
_S = _B * _BAG
_NC, _NS = 2, 16
_NW = _NC * _NS
_SW = _S // _NW


def _embed_sum(table_w, sent_t):
    mesh = plsc.VectorSubcoreMesh(core_axis_name="c", subcore_axis_name="s")

    @functools.partial(
        pl.kernel, mesh=mesh,
        out_type=jax.ShapeDtypeStruct((_S, 2 * _D), jnp.float32),
        scratch_types=[
            pltpu.VMEM((_L, _SW), jnp.int32),
            pltpu.VMEM((_SW, 2 * _D), jnp.float32),
            pltpu.SemaphoreType.DMA,
        ],
    )
    def k(table_hbm, sent_hbm, out_hbm, idx_v, acc_v, sem):
        cid = lax.axis_index("c")
        sid = lax.axis_index("s")
        wid = sid * _NC + cid

        pltpu.sync_copy(sent_hbm.at[wid], idx_v)

        zeros = jnp.zeros((16,), jnp.float32)

        def zbody(i, carry):
            for j in range(2 * _D // 16):
                acc_v[i, pl.ds(j * 16, 16)] = zeros
            return carry
        lax.fori_loop(0, _SW, zbody, 0)

        copies = [
            pltpu.async_copy(table_hbm.at[idx_v.at[j]], acc_v, sem, add=True)
            for j in range(_L)
        ]
        for c in copies:
            c.wait()

        pltpu.sync_copy(acc_v, out_hbm.at[pl.ds(wid * _SW, _SW)])

    return k(table_w, sent_t)


def _mlp_body(bag_ref, mr_ref, w1a_ref, w1b_ref, b1_ref, w2_ref, b2_ref,
              o_ref):
    h = jnp.dot(bag_ref[:, :_D], w1a_ref[...],
                preferred_element_type=jnp.float32)
    h = h + jnp.dot(mr_ref[...], w1b_ref[...],
                    preferred_element_type=jnp.float32)
    h = jnp.maximum(h + b1_ref[...], 0.0)
    o = jnp.dot(h, w2_ref[...], preferred_element_type=jnp.float32)
    o_ref[...] = jnp.maximum(o + b2_ref[...], 0.0)


def _mlp(bag, mr, w1a, w1b, b1, w2, b2):
    R = 512
    return pl.pallas_call(
        _mlp_body,
        grid=(_S // R,),
        in_specs=[
            pl.BlockSpec((R, 2 * _D), lambda i: (i, 0)),
            pl.BlockSpec((R, _MD), lambda i: (i, 0)),
            pl.BlockSpec((_D, 2 * _H), lambda i: (0, 0)),
            pl.BlockSpec((_MD, 2 * _H), lambda i: (0, 0)),
            pl.BlockSpec((1, 2 * _H), lambda i: (0, 0)),
            pl.BlockSpec((2 * _H, _H), lambda i: (0, 0)),
            pl.BlockSpec((1, _H), lambda i: (0, 0)),
        ],
        out_specs=pl.BlockSpec((R, _H), lambda i: (i, 0)),
        out_shape=jax.ShapeDtypeStruct((_S, _H), jnp.float32),
    )(bag, mr, w1a, w1b, b1, w2, b2)


def kernel(sentences, mention_rep, table, W1, b1, W2, b2):
    table_w = jnp.pad(table, ((0, 0), (0, _D)))
    sent_t = sentences.reshape(_NW, _SW, _L).transpose(0, 2, 1)
    bag_sum = _embed_sum(table_w, sent_t)
    w1a = W1[:_D] * jnp.float32(1.0 / _L)
    w1b = W1[_D:]
    mr = mention_rep.reshape(_S, _MD)
    return _mlp(bag_sum, mr, w1a, w1b, b1.reshape(1, 2 * _H), W2,
                b2.reshape(1, _H))
